# baseline (speedup 1.0000x reference)
import functools

import jax
import jax.numpy as jnp
from jax import lax
from jax.experimental import pallas as pl
from jax.experimental.pallas import tpu as pltpu
from jax.experimental.pallas import tpu_sc as plsc

N = 10000
E = 320000
D_IN = 128
H = 256
HH = 128
G = 16
OUT = 16
FC = 1024
NT = 16
SLAB = 624
TAIL = N - NT * SLAB
CH = 80
SP_CHUNKS = (E // NT) // CH
GB = 10
NG = SP_CHUNKS // GB
CHD = 80
GBD = 25
NGD = (E // (2 * NT)) // (GBD * CHD)
ZR = 78
RB = 1000
GRID = N // RB
_F32 = jnp.float32


def _degk_body(dstd, dega, degb, dst_v, ones_v, zero_v, ssem, acc_sh):
    c = lax.axis_index("c")
    s = lax.axis_index("s")
    for i in range(CHD // 16):
        ones_v[pl.ds(i * 16, 16)] = jnp.ones((16,), _F32)
    for i in range(SLAB // 16):
        zero_v[pl.ds(i * 16, 16)] = jnp.zeros((16,), _F32)
    sl = pl.ds(s * SLAB, SLAB)
    tl = pl.ds(NT * SLAB, TAIL)
    pltpu.sync_copy(zero_v, acc_sh.at[sl])

    @pl.when(s == NT - 1)
    def _():
        pltpu.sync_copy(zero_v.at[pl.ds(0, TAIL)], acc_sh.at[tl])

    plsc.subcore_barrier()

    def group(g, carry):
        pltpu.sync_copy(dstd.at[c, s, g], dst_v)
        shs = [pltpu.async_copy(ones_v, acc_sh.at[dst_v.at[j]], ssem,
                                add=True)
               for j in range(GBD)]
        for h in shs:
            h.wait()
        return carry

    lax.fori_loop(0, NGD, group, 0)
    plsc.subcore_barrier()

    pltpu.sync_copy(acc_sh.at[sl], zero_v)

    @pl.when(c == 0)
    def _():
        pltpu.sync_copy(zero_v, dega.at[sl])

    @pl.when(c == 1)
    def _():
        pltpu.sync_copy(zero_v, degb.at[sl])

    @pl.when(s == NT - 1)
    def _():
        pltpu.sync_copy(acc_sh.at[tl], ones_v.at[pl.ds(0, TAIL)])

        @pl.when(c == 0)
        def _():
            pltpu.sync_copy(ones_v.at[pl.ds(0, TAIL)], dega.at[tl])

        @pl.when(c == 1)
        def _():
            pltpu.sync_copy(ones_v.at[pl.ds(0, TAIL)], degb.at[tl])


@functools.cache
def _get_degk():
    return pl.kernel(
        _degk_body,
        out_type=[jax.ShapeDtypeStruct((N,), _F32),
                  jax.ShapeDtypeStruct((N,), _F32)],
        mesh=plsc.VectorSubcoreMesh(core_axis_name="c", subcore_axis_name="s"),
        scratch_types=[
            pltpu.VMEM((GBD, CHD), jnp.int32),
            pltpu.VMEM((CHD,), _F32),
            pltpu.VMEM((SLAB,), _F32),
            pltpu.SemaphoreType.DMA,
            pltpu.VMEM_SHARED((N,), _F32),
        ],
    )


NB = 3


def _spmm_body(ylo, yhi, src3, dst3, alo, ahi, src_v, dst_v,
               rows, gsems, ssems, isems, acc_sh):
    c = lax.axis_index("c")
    s = lax.axis_index("s")
    sl = pl.ds(s * SLAB, SLAB)
    tl = pl.ds(NT * SLAB, TAIL)

    def run(y_hbm, out_hbm):
        pltpu.sync_copy(y_hbm.at[sl], acc_sh.at[sl])

        @pl.when(s == NT - 1)
        def _():
            pltpu.sync_copy(y_hbm.at[tl], acc_sh.at[tl])

        plsc.subcore_barrier()

        def gather(gm, j):
            return pltpu.async_copy(
                y_hbm.at[src_v.at[gm, j]], rows.at[j % NB], gsems.at[j % NB])

        def scatter(gm, j):
            return pltpu.async_copy(
                rows.at[j % NB], acc_sh.at[dst_v.at[gm, j]], ssems.at[j % NB],
                add=True)

        pltpu.sync_copy(src3.at[s, 0], src_v.at[0])
        pltpu.sync_copy(dst3.at[s, 0], dst_v.at[0])

        def group(g, carry):
            gm = lax.rem(g, 2)
            gn = lax.rem(g + 1, 2)

            @pl.when(g + 1 < NG)
            def _():
                pltpu.async_copy(src3.at[s, g + 1], src_v.at[gn], isems.at[0])
                pltpu.async_copy(dst3.at[s, g + 1], dst_v.at[gn], isems.at[1])

            ghs = [None] * GB
            shs = [None] * GB
            ghs[0] = gather(gm, 0)
            for j in range(GB):
                if j >= 2:
                    shs[j - 2].wait()
                if j + 1 < GB:
                    ghs[j + 1] = gather(gm, j + 1)
                ghs[j].wait()
                shs[j] = scatter(gm, j)
            for j in range(GB - 2, GB):
                shs[j].wait()

            @pl.when(g + 1 < NG)
            def _():
                pltpu.make_async_copy(src3.at[s, g + 1], src_v.at[gn],
                                      isems.at[0]).wait()
                pltpu.make_async_copy(dst3.at[s, g + 1], dst_v.at[gn],
                                      isems.at[1]).wait()

            return carry

        lax.fori_loop(0, NG, group, 0)
        plsc.subcore_barrier()
        pltpu.sync_copy(acc_sh.at[sl], out_hbm.at[sl])

        @pl.when(s == NT - 1)
        def _():
            pltpu.sync_copy(acc_sh.at[tl], out_hbm.at[tl])

    @pl.when(c == 0)
    def _():
        run(ylo, alo)

    @pl.when(c == 1)
    def _():
        run(yhi, ahi)


@functools.cache
def _get_spmm():
    return pl.kernel(
        _spmm_body,
        out_type=[jax.ShapeDtypeStruct((N, HH), _F32),
                  jax.ShapeDtypeStruct((N, HH), _F32)],
        mesh=plsc.VectorSubcoreMesh(core_axis_name="c", subcore_axis_name="s"),
        scratch_types=[
            pltpu.VMEM((2, GB, CH), jnp.int32),
            pltpu.VMEM((2, GB, CH), jnp.int32),
            pltpu.VMEM((NB, CH, HH), _F32),
            pltpu.SemaphoreType.DMA((NB,)),
            pltpu.SemaphoreType.DMA((NB,)),
            pltpu.SemaphoreType.DMA((2,)),
            pltpu.VMEM_SHARED((N, HH), _F32),
        ],
    )


def _mm0_body(x, w, xwlo, xwhi):
    xw = jnp.dot(x[...], w[...], preferred_element_type=_F32, precision=lax.Precision.HIGHEST)
    xwlo[...] = xw[:, :HH]
    xwhi[...] = xw[:, HH:]


_mm0 = pl.pallas_call(
    _mm0_body,
    grid=(GRID,),
    in_specs=[
        pl.BlockSpec((RB, D_IN), lambda i: (i, 0)),
        pl.BlockSpec((D_IN, H), lambda i: (0, 0)),
    ],
    out_specs=[
        pl.BlockSpec((RB, HH), lambda i: (i, 0)),
        pl.BlockSpec((RB, HH), lambda i: (i, 0)),
    ],
    out_shape=[
        jax.ShapeDtypeStruct((N, HH), _F32),
        jax.ShapeDtypeStruct((N, HH), _F32),
    ],
)


def _scale_body(dega, degb, xwlo, xwhi, ylo, yhi, dinv_ref):
    dinv = lax.rsqrt(dega[...] + degb[...] + 1.0)
    ylo[...] = dinv * xwlo[...]
    yhi[...] = dinv * xwhi[...]
    dinv_ref[...] = dinv


_scale = pl.pallas_call(
    _scale_body,
    grid=(GRID,),
    in_specs=[
        pl.BlockSpec((RB, 1), lambda i: (i, 0)),
        pl.BlockSpec((RB, 1), lambda i: (i, 0)),
        pl.BlockSpec((RB, HH), lambda i: (i, 0)),
        pl.BlockSpec((RB, HH), lambda i: (i, 0)),
    ],
    out_specs=[
        pl.BlockSpec((RB, HH), lambda i: (i, 0)),
        pl.BlockSpec((RB, HH), lambda i: (i, 0)),
        pl.BlockSpec((RB, 1), lambda i: (i, 0)),
    ],
    out_shape=[
        jax.ShapeDtypeStruct((N, HH), _F32),
        jax.ShapeDtypeStruct((N, HH), _F32),
        jax.ShapeDtypeStruct((N, 1), _F32),
    ],
)


def _post_body(alo, ahi, dinv, b, z_ref, s_ref, ss_ref):
    i = pl.program_id(0)
    acc = jnp.concatenate([alo[...], ahi[...]], axis=1)
    z = jnp.maximum(dinv[...] * acc + b[...], 0.0)
    z_ref[...] = z

    @pl.when(i == 0)
    def _():
        s_ref[...] = jnp.zeros_like(s_ref)
        ss_ref[...] = jnp.zeros_like(ss_ref)

    s_ref[...] += jnp.sum(z, axis=0, keepdims=True)
    ss_ref[...] += jnp.sum(z * z, axis=0, keepdims=True)


_post = pl.pallas_call(
    _post_body,
    grid=(GRID,),
    in_specs=[
        pl.BlockSpec((RB, HH), lambda i: (i, 0)),
        pl.BlockSpec((RB, HH), lambda i: (i, 0)),
        pl.BlockSpec((RB, 1), lambda i: (i, 0)),
        pl.BlockSpec((H,), lambda i: (0,)),
    ],
    out_specs=[
        pl.BlockSpec((RB, H), lambda i: (i, 0)),
        pl.BlockSpec((1, H), lambda i: (0, 0)),
        pl.BlockSpec((1, H), lambda i: (0, 0)),
    ],
    out_shape=[
        jax.ShapeDtypeStruct((N, H), _F32),
        jax.ShapeDtypeStruct((1, H), _F32),
        jax.ShapeDtypeStruct((1, H), _F32),
    ],
)


def _postp_body(alo, ahi, dinv, b, bidx, s_ref, ss_ref, feat_ref, cnt_ref):
    i = pl.program_id(0)
    acc = jnp.concatenate([alo[...], ahi[...]], axis=1)
    z = jnp.maximum(dinv[...] * acc + b[...], 0.0)
    bi = bidx[0]
    row = jnp.broadcast_to(bi, (G, RB))
    col = lax.broadcasted_iota(jnp.int32, (G, RB), 0)
    oht = (row == col).astype(_F32)

    @pl.when(i == 0)
    def _():
        s_ref[...] = jnp.zeros_like(s_ref)
        ss_ref[...] = jnp.zeros_like(ss_ref)
        feat_ref[...] = jnp.zeros_like(feat_ref)
        cnt_ref[...] = jnp.zeros_like(cnt_ref)

    s_ref[...] += jnp.sum(z, axis=0, keepdims=True)
    ss_ref[...] += jnp.sum(z * z, axis=0, keepdims=True)
    feat_ref[...] += jnp.dot(oht, z, preferred_element_type=_F32,
                             precision=lax.Precision.HIGHEST)
    cnt_ref[...] += jnp.broadcast_to(jnp.sum(oht, axis=1, keepdims=True), (G, H))


_postp = pl.pallas_call(
    _postp_body,
    grid=(GRID,),
    in_specs=[
        pl.BlockSpec((RB, HH), lambda i: (i, 0)),
        pl.BlockSpec((RB, HH), lambda i: (i, 0)),
        pl.BlockSpec((RB, 1), lambda i: (i, 0)),
        pl.BlockSpec((H,), lambda i: (0,)),
        pl.BlockSpec((1, 1, RB), lambda i: (i, 0, 0)),
    ],
    out_specs=[
        pl.BlockSpec((1, H), lambda i: (0, 0)),
        pl.BlockSpec((1, H), lambda i: (0, 0)),
        pl.BlockSpec((G, H), lambda i: (0, 0)),
        pl.BlockSpec((G, H), lambda i: (0, 0)),
    ],
    out_shape=[
        jax.ShapeDtypeStruct((1, H), _F32),
        jax.ShapeDtypeStruct((1, H), _F32),
        jax.ShapeDtypeStruct((G, H), _F32),
        jax.ShapeDtypeStruct((G, H), _F32),
    ],
)


def _nm_body(z, s, ss, g, be, w, dinv, ylo, yhi):
    m = s[...] * (1.0 / N)
    v = ss[...] * (1.0 / N) - m * m
    a = g[...] * lax.rsqrt(v + 1e-5)
    zn = (z[...] - m) * a + be[...]
    y = dinv[...] * jnp.dot(zn, w[...], preferred_element_type=_F32, precision=lax.Precision.HIGHEST)
    ylo[...] = y[:, :HH]
    yhi[...] = y[:, HH:]


_nm = pl.pallas_call(
    _nm_body,
    grid=(GRID,),
    in_specs=[
        pl.BlockSpec((RB, H), lambda i: (i, 0)),
        pl.BlockSpec((1, H), lambda i: (0, 0)),
        pl.BlockSpec((1, H), lambda i: (0, 0)),
        pl.BlockSpec((H,), lambda i: (0,)),
        pl.BlockSpec((H,), lambda i: (0,)),
        pl.BlockSpec((H, H), lambda i: (0, 0)),
        pl.BlockSpec((RB, 1), lambda i: (i, 0)),
    ],
    out_specs=[
        pl.BlockSpec((RB, HH), lambda i: (i, 0)),
        pl.BlockSpec((RB, HH), lambda i: (i, 0)),
    ],
    out_shape=[
        jax.ShapeDtypeStruct((N, HH), _F32),
        jax.ShapeDtypeStruct((N, HH), _F32),
    ],
)


def _inorm_relu(t):
    m = jnp.mean(t, axis=-1, keepdims=True)
    v = jnp.mean((t - m) * (t - m), axis=-1, keepdims=True)
    return jnp.maximum((t - m) * lax.rsqrt(v + 1e-5), 0.0)


def _head_body(fs, cnt, s, ss, g, be, fw1, fb1, fw2, fb2, fw3, fb3, out_ref):
    m = s[...] * (1.0 / N)
    v = ss[...] * (1.0 / N) - m * m
    a = g[...] * lax.rsqrt(v + 1e-5)
    cnt = cnt[...]
    fsn = (fs[...] - cnt * m) * a + cnt * be[...]
    feat = fsn / jnp.maximum(cnt, 1.0)
    t = jnp.dot(feat, fw1[...], preferred_element_type=_F32, precision=lax.Precision.HIGHEST) + fb1[...]
    t = _inorm_relu(t)
    t = jnp.dot(t, fw2[...], preferred_element_type=_F32, precision=lax.Precision.HIGHEST) + fb2[...]
    t = _inorm_relu(t)
    out_ref[...] = jnp.dot(t, fw3[...], preferred_element_type=_F32, precision=lax.Precision.HIGHEST) + fb3[...]


_head = pl.pallas_call(
    _head_body,
    out_shape=jax.ShapeDtypeStruct((G, OUT), _F32),
)


def kernel(x, edge_index, batch_idx, W1, b1, g1, be1, W2, b2, g2, be2,
           W3, b3, g3, be3, fW1, fb1, fW2, fb2, fW3, fb3):
    src3 = edge_index[0].reshape(NT, NG, GB, CH)
    dst3 = edge_index[1].reshape(NT, NG, GB, CH)
    dstd = edge_index[1].reshape(2, NT, NGD, GBD, CHD)
    bidx3 = batch_idx.reshape(GRID, 1, RB)

    _spmm = _get_spmm()

    dega, degb = _get_degk()(dstd)
    xwlo, xwhi = _mm0(x, W1)
    ylo, yhi, dinv = _scale(dega.reshape(N, 1), degb.reshape(N, 1), xwlo, xwhi)

    alo, ahi = _spmm(ylo, yhi, src3, dst3)
    z, s, ss = _post(alo, ahi, dinv, b1)
    ylo, yhi = _nm(z, s, ss, g1, be1, W2, dinv)

    alo, ahi = _spmm(ylo, yhi, src3, dst3)
    z, s, ss = _post(alo, ahi, dinv, b2)
    ylo, yhi = _nm(z, s, ss, g2, be2, W3, dinv)

    alo, ahi = _spmm(ylo, yhi, src3, dst3)
    s, ss, feat, cnt = _postp(alo, ahi, dinv, b3, bidx3)

    return _head(feat, cnt, s, ss, g3, be3, fW1, fb1, fW2, fb2, fW3, fb3)


# speedup vs baseline: 1.0131x; 1.0131x over previous
import functools

import jax
import jax.numpy as jnp
from jax import lax
from jax.experimental import pallas as pl
from jax.experimental.pallas import tpu as pltpu
from jax.experimental.pallas import tpu_sc as plsc

N = 10000
E = 320000
D_IN = 128
H = 256
HH = 128
G = 16
OUT = 16
FC = 1024
NT = 16
SLAB = 624
TAIL = N - NT * SLAB
CH = 80
SP_CHUNKS = (E // NT) // CH
GB = 10
NG = SP_CHUNKS // GB
CHD = 80
GBD = 25
NGD = (E // (2 * NT)) // (GBD * CHD)
ZR = 78
RB = 1000
GRID = N // RB
_F32 = jnp.float32


def _degk_body(dstd, dega, degb, dst_v, ones_v, zero_v, ssem, acc_sh):
    c = lax.axis_index("c")
    s = lax.axis_index("s")
    for i in range(CHD // 16):
        ones_v[pl.ds(i * 16, 16)] = jnp.ones((16,), _F32)
    for i in range(SLAB // 16):
        zero_v[pl.ds(i * 16, 16)] = jnp.zeros((16,), _F32)
    sl = pl.ds(s * SLAB, SLAB)
    tl = pl.ds(NT * SLAB, TAIL)
    pltpu.sync_copy(zero_v, acc_sh.at[sl])

    @pl.when(s == NT - 1)
    def _():
        pltpu.sync_copy(zero_v.at[pl.ds(0, TAIL)], acc_sh.at[tl])

    plsc.subcore_barrier()

    def group(g, carry):
        pltpu.sync_copy(dstd.at[c, s, g], dst_v)
        shs = [pltpu.async_copy(ones_v, acc_sh.at[dst_v.at[j]], ssem,
                                add=True)
               for j in range(GBD)]
        for h in shs:
            h.wait()
        return carry

    lax.fori_loop(0, NGD, group, 0)
    plsc.subcore_barrier()

    pltpu.sync_copy(acc_sh.at[sl], zero_v)

    @pl.when(c == 0)
    def _():
        pltpu.sync_copy(zero_v, dega.at[sl])

    @pl.when(c == 1)
    def _():
        pltpu.sync_copy(zero_v, degb.at[sl])

    @pl.when(s == NT - 1)
    def _():
        pltpu.sync_copy(acc_sh.at[tl], ones_v.at[pl.ds(0, TAIL)])

        @pl.when(c == 0)
        def _():
            pltpu.sync_copy(ones_v.at[pl.ds(0, TAIL)], dega.at[tl])

        @pl.when(c == 1)
        def _():
            pltpu.sync_copy(ones_v.at[pl.ds(0, TAIL)], degb.at[tl])


@functools.cache
def _get_degk():
    return pl.kernel(
        _degk_body,
        out_type=[jax.ShapeDtypeStruct((N,), _F32),
                  jax.ShapeDtypeStruct((N,), _F32)],
        mesh=plsc.VectorSubcoreMesh(core_axis_name="c", subcore_axis_name="s"),
        scratch_types=[
            pltpu.VMEM((GBD, CHD), jnp.int32),
            pltpu.VMEM((CHD,), _F32),
            pltpu.VMEM((SLAB,), _F32),
            pltpu.SemaphoreType.DMA,
            pltpu.VMEM_SHARED((N,), _F32),
        ],
    )


NB = 3


def _spmm_body(ylo, yhi, src3, dst3, alo, ahi, src_v, dst_v,
               rows, gsems, ssems, isems, acc_sh):
    c = lax.axis_index("c")
    s = lax.axis_index("s")
    sl = pl.ds(s * SLAB, SLAB)
    tl = pl.ds(NT * SLAB, TAIL)

    def run(y_hbm, out_hbm):
        pltpu.sync_copy(y_hbm.at[sl], acc_sh.at[sl])

        @pl.when(s == NT - 1)
        def _():
            pltpu.sync_copy(y_hbm.at[tl], acc_sh.at[tl])

        plsc.subcore_barrier()

        def gather(gm, j):
            return pltpu.async_copy(
                y_hbm.at[src_v.at[gm, j]], rows.at[j % NB], gsems.at[j % NB])

        def scatter(gm, j):
            return pltpu.async_copy(
                rows.at[j % NB], acc_sh.at[dst_v.at[gm, j]], ssems.at[j % NB],
                add=True)

        pltpu.sync_copy(src3.at[s, 0], src_v.at[0])
        pltpu.sync_copy(dst3.at[s, 0], dst_v.at[0])

        def group(g, carry):
            gm = lax.rem(g, 2)
            gn = lax.rem(g + 1, 2)

            @pl.when(g + 1 < NG)
            def _():
                pltpu.async_copy(src3.at[s, g + 1], src_v.at[gn], isems.at[0])
                pltpu.async_copy(dst3.at[s, g + 1], dst_v.at[gn], isems.at[1])

            ghs = [None] * GB
            shs = [None] * GB
            ghs[0] = gather(gm, 0)
            for j in range(GB):
                if j >= 2:
                    shs[j - 2].wait()
                if j + 1 < GB:
                    ghs[j + 1] = gather(gm, j + 1)
                ghs[j].wait()
                shs[j] = scatter(gm, j)
            for j in range(GB - 2, GB):
                shs[j].wait()

            @pl.when(g + 1 < NG)
            def _():
                pltpu.make_async_copy(src3.at[s, g + 1], src_v.at[gn],
                                      isems.at[0]).wait()
                pltpu.make_async_copy(dst3.at[s, g + 1], dst_v.at[gn],
                                      isems.at[1]).wait()

            return carry

        lax.fori_loop(0, NG, group, 0)
        plsc.subcore_barrier()
        pltpu.sync_copy(acc_sh.at[sl], out_hbm.at[sl])

        @pl.when(s == NT - 1)
        def _():
            pltpu.sync_copy(acc_sh.at[tl], out_hbm.at[tl])

    @pl.when(c == 0)
    def _():
        run(ylo, alo)

    @pl.when(c == 1)
    def _():
        run(yhi, ahi)


@functools.cache
def _get_spmm():
    return pl.kernel(
        _spmm_body,
        out_type=[jax.ShapeDtypeStruct((N, HH), _F32),
                  jax.ShapeDtypeStruct((N, HH), _F32)],
        mesh=plsc.VectorSubcoreMesh(core_axis_name="c", subcore_axis_name="s"),
        scratch_types=[
            pltpu.VMEM((2, GB, CH), jnp.int32),
            pltpu.VMEM((2, GB, CH), jnp.int32),
            pltpu.VMEM((NB, CH, HH), _F32),
            pltpu.SemaphoreType.DMA((NB,)),
            pltpu.SemaphoreType.DMA((NB,)),
            pltpu.SemaphoreType.DMA((2,)),
            pltpu.VMEM_SHARED((N, HH), _F32),
        ],
    )


def _mm1_body(dega, degb, x, w, ylo, yhi, dinv_ref):
    dinv = lax.rsqrt(dega[...] + degb[...] + 1.0)
    y = dinv * jnp.dot(x[...], w[...], preferred_element_type=_F32, precision=lax.Precision.HIGHEST)
    ylo[...] = y[:, :HH]
    yhi[...] = y[:, HH:]
    dinv_ref[...] = dinv


_mm1 = pl.pallas_call(
    _mm1_body,
    grid=(GRID,),
    in_specs=[
        pl.BlockSpec((RB, 1), lambda i: (i, 0)),
        pl.BlockSpec((RB, 1), lambda i: (i, 0)),
        pl.BlockSpec((RB, D_IN), lambda i: (i, 0)),
        pl.BlockSpec((D_IN, H), lambda i: (0, 0)),
    ],
    out_specs=[
        pl.BlockSpec((RB, HH), lambda i: (i, 0)),
        pl.BlockSpec((RB, HH), lambda i: (i, 0)),
        pl.BlockSpec((RB, 1), lambda i: (i, 0)),
    ],
    out_shape=[
        jax.ShapeDtypeStruct((N, HH), _F32),
        jax.ShapeDtypeStruct((N, HH), _F32),
        jax.ShapeDtypeStruct((N, 1), _F32),
    ],
)


def _post_body(alo, ahi, dinv, b, s_ref, ss_ref):
    i = pl.program_id(0)
    acc = jnp.concatenate([alo[...], ahi[...]], axis=1)
    z = jnp.maximum(dinv[...] * acc + b[...], 0.0)

    @pl.when(i == 0)
    def _():
        s_ref[...] = jnp.zeros_like(s_ref)
        ss_ref[...] = jnp.zeros_like(ss_ref)

    s_ref[...] += jnp.sum(z, axis=0, keepdims=True)
    ss_ref[...] += jnp.sum(z * z, axis=0, keepdims=True)


_post = pl.pallas_call(
    _post_body,
    grid=(GRID,),
    in_specs=[
        pl.BlockSpec((RB, HH), lambda i: (i, 0)),
        pl.BlockSpec((RB, HH), lambda i: (i, 0)),
        pl.BlockSpec((RB, 1), lambda i: (i, 0)),
        pl.BlockSpec((H,), lambda i: (0,)),
    ],
    out_specs=[
        pl.BlockSpec((1, H), lambda i: (0, 0)),
        pl.BlockSpec((1, H), lambda i: (0, 0)),
    ],
    out_shape=[
        jax.ShapeDtypeStruct((1, H), _F32),
        jax.ShapeDtypeStruct((1, H), _F32),
    ],
)


def _postp_body(alo, ahi, dinv, b, bidx, s_ref, ss_ref, feat_ref, cnt_ref):
    i = pl.program_id(0)
    acc = jnp.concatenate([alo[...], ahi[...]], axis=1)
    z = jnp.maximum(dinv[...] * acc + b[...], 0.0)
    bi = bidx[0]
    row = jnp.broadcast_to(bi, (G, RB))
    col = lax.broadcasted_iota(jnp.int32, (G, RB), 0)
    oht = (row == col).astype(_F32)

    @pl.when(i == 0)
    def _():
        s_ref[...] = jnp.zeros_like(s_ref)
        ss_ref[...] = jnp.zeros_like(ss_ref)
        feat_ref[...] = jnp.zeros_like(feat_ref)
        cnt_ref[...] = jnp.zeros_like(cnt_ref)

    s_ref[...] += jnp.sum(z, axis=0, keepdims=True)
    ss_ref[...] += jnp.sum(z * z, axis=0, keepdims=True)
    feat_ref[...] += jnp.dot(oht, z, preferred_element_type=_F32,
                             precision=lax.Precision.HIGHEST)
    cnt_ref[...] += jnp.broadcast_to(jnp.sum(oht, axis=1, keepdims=True), (G, H))


_postp = pl.pallas_call(
    _postp_body,
    grid=(GRID,),
    in_specs=[
        pl.BlockSpec((RB, HH), lambda i: (i, 0)),
        pl.BlockSpec((RB, HH), lambda i: (i, 0)),
        pl.BlockSpec((RB, 1), lambda i: (i, 0)),
        pl.BlockSpec((H,), lambda i: (0,)),
        pl.BlockSpec((1, 1, RB), lambda i: (i, 0, 0)),
    ],
    out_specs=[
        pl.BlockSpec((1, H), lambda i: (0, 0)),
        pl.BlockSpec((1, H), lambda i: (0, 0)),
        pl.BlockSpec((G, H), lambda i: (0, 0)),
        pl.BlockSpec((G, H), lambda i: (0, 0)),
    ],
    out_shape=[
        jax.ShapeDtypeStruct((1, H), _F32),
        jax.ShapeDtypeStruct((1, H), _F32),
        jax.ShapeDtypeStruct((G, H), _F32),
        jax.ShapeDtypeStruct((G, H), _F32),
    ],
)


def _nm_body(alo, ahi, b, s, ss, g, be, w, dinv, ylo, yhi):
    acc = jnp.concatenate([alo[...], ahi[...]], axis=1)
    z = jnp.maximum(dinv[...] * acc + b[...], 0.0)
    m = s[...] * (1.0 / N)
    v = ss[...] * (1.0 / N) - m * m
    a = g[...] * lax.rsqrt(v + 1e-5)
    zn = (z - m) * a + be[...]
    y = dinv[...] * jnp.dot(zn, w[...], preferred_element_type=_F32, precision=lax.Precision.HIGHEST)
    ylo[...] = y[:, :HH]
    yhi[...] = y[:, HH:]


_nm = pl.pallas_call(
    _nm_body,
    grid=(GRID,),
    in_specs=[
        pl.BlockSpec((RB, HH), lambda i: (i, 0)),
        pl.BlockSpec((RB, HH), lambda i: (i, 0)),
        pl.BlockSpec((H,), lambda i: (0,)),
        pl.BlockSpec((1, H), lambda i: (0, 0)),
        pl.BlockSpec((1, H), lambda i: (0, 0)),
        pl.BlockSpec((H,), lambda i: (0,)),
        pl.BlockSpec((H,), lambda i: (0,)),
        pl.BlockSpec((H, H), lambda i: (0, 0)),
        pl.BlockSpec((RB, 1), lambda i: (i, 0)),
    ],
    out_specs=[
        pl.BlockSpec((RB, HH), lambda i: (i, 0)),
        pl.BlockSpec((RB, HH), lambda i: (i, 0)),
    ],
    out_shape=[
        jax.ShapeDtypeStruct((N, HH), _F32),
        jax.ShapeDtypeStruct((N, HH), _F32),
    ],
)


def _inorm_relu(t):
    m = jnp.mean(t, axis=-1, keepdims=True)
    v = jnp.mean((t - m) * (t - m), axis=-1, keepdims=True)
    return jnp.maximum((t - m) * lax.rsqrt(v + 1e-5), 0.0)


def _head_body(fs, cnt, s, ss, g, be, fw1, fb1, fw2, fb2, fw3, fb3, out_ref):
    m = s[...] * (1.0 / N)
    v = ss[...] * (1.0 / N) - m * m
    a = g[...] * lax.rsqrt(v + 1e-5)
    cnt = cnt[...]
    fsn = (fs[...] - cnt * m) * a + cnt * be[...]
    feat = fsn / jnp.maximum(cnt, 1.0)
    t = jnp.dot(feat, fw1[...], preferred_element_type=_F32, precision=lax.Precision.HIGHEST) + fb1[...]
    t = _inorm_relu(t)
    t = jnp.dot(t, fw2[...], preferred_element_type=_F32, precision=lax.Precision.HIGHEST) + fb2[...]
    t = _inorm_relu(t)
    out_ref[...] = jnp.dot(t, fw3[...], preferred_element_type=_F32, precision=lax.Precision.HIGHEST) + fb3[...]


_head = pl.pallas_call(
    _head_body,
    out_shape=jax.ShapeDtypeStruct((G, OUT), _F32),
)


def kernel(x, edge_index, batch_idx, W1, b1, g1, be1, W2, b2, g2, be2,
           W3, b3, g3, be3, fW1, fb1, fW2, fb2, fW3, fb3):
    src3 = edge_index[0].reshape(NT, NG, GB, CH)
    dst3 = edge_index[1].reshape(NT, NG, GB, CH)
    dstd = edge_index[1].reshape(2, NT, NGD, GBD, CHD)
    bidx3 = batch_idx.reshape(GRID, 1, RB)

    _spmm = _get_spmm()

    dega, degb = _get_degk()(dstd)
    ylo, yhi, dinv = _mm1(dega.reshape(N, 1), degb.reshape(N, 1), x, W1)

    alo, ahi = _spmm(ylo, yhi, src3, dst3)
    s, ss = _post(alo, ahi, dinv, b1)
    ylo, yhi = _nm(alo, ahi, b1, s, ss, g1, be1, W2, dinv)

    alo, ahi = _spmm(ylo, yhi, src3, dst3)
    s, ss = _post(alo, ahi, dinv, b2)
    ylo, yhi = _nm(alo, ahi, b2, s, ss, g2, be2, W3, dinv)

    alo, ahi = _spmm(ylo, yhi, src3, dst3)
    s, ss, feat, cnt = _postp(alo, ahi, dinv, b3, bidx3)

    return _head(feat, cnt, s, ss, g3, be3, fW1, fb1, fW2, fb2, fW3, fb3)
